# baseline (speedup 1.0000x reference)
import functools

import jax
import jax.numpy as jnp
from jax import lax
from jax.experimental import pallas as pl
from jax.experimental.pallas import tpu as pltpu
from jax.experimental.pallas import tpu_sc as plsc

_B, _T, _D = 16, 1024, 32
_K, _CD = 8192, 32
_N = _B * _T
_M = 512
_EPS = 1e-12

_CB = 8
_CT = _CB * _T
_NBLKC = _CT // _M
_TPB = _T // _M

_NC, _NS = 2, 16
_NW = _NC * _NS
_BPW = _CT // _NW


def _argmin_body(z_ref, v_ref, g_ref, b_ref, fc_ref,
                 idx_ref, cbp_ref, cb_ref, c2_ref):
    i = pl.program_id(0)

    @pl.when(i == 0)
    def _init():
        v = v_ref[...]
        vn = jnp.sqrt(jnp.sum(v * v, axis=1, keepdims=True))
        w = g_ref[...].reshape(_D, 1) * v / vn
        cb = jnp.dot(fc_ref[...], w.T,
                     preferred_element_type=jnp.float32) + b_ref[...]
        cb_ref[...] = cb
        c2 = jnp.sum(cb * cb, axis=1).reshape(1, _K)
        c2_ref[...] = jnp.broadcast_to(c2, (_M, _K))
        if cbp_ref is not None:
            cbp_ref[...] = jnp.pad(cb, ((0, 0), (0, 128 - _D)))

    cb = cb_ref[...]
    z = z_ref[...].reshape(_M, _D)
    z2 = jnp.sum(z * z, axis=1, keepdims=True)
    ncross = jnp.dot(-2.0 * z, cb.T,
                     preferred_element_type=jnp.float32)
    d2 = z2 + ncross + c2_ref[...]
    idx = jnp.argmin(d2, axis=1).astype(jnp.int32)
    idx_ref[...] = idx.reshape(1, 1, _M)


def _argmin_step_cbp(z_ref, v_ref, g_ref, b_ref, fc_ref,
                     idx_ref, cbp_ref, cb_ref, c2_ref):
    _argmin_body(z_ref, v_ref, g_ref, b_ref, fc_ref,
                 idx_ref, cbp_ref, cb_ref, c2_ref)


def _argmin_step(z_ref, v_ref, g_ref, b_ref, fc_ref,
                 idx_ref, cb_ref, c2_ref):
    _argmin_body(z_ref, v_ref, g_ref, b_ref, fc_ref,
                 idx_ref, None, cb_ref, c2_ref)


def _gather_sc(cbp_hbm, idx_hbm, out_hbm, idx_v, rows_v, sem):
    wid = lax.axis_index("s") * _NC + lax.axis_index("c")
    base = wid * _BPW
    pltpu.sync_copy(idx_hbm.at[pl.ds(base, _BPW)], idx_v)
    pltpu.async_copy(cbp_hbm.at[idx_v], rows_v, sem).wait()
    pltpu.sync_copy(rows_v, out_hbm.at[pl.ds(base, _BPW)])


def _rotate_step(z_ref, zq_ref, out_ref, loss_ref):
    z = z_ref[...].reshape(_CT, _D)
    zq = zq_ref[:, :_D]
    diff = z - zq
    loss_ref[...] = jnp.sum(diff * diff).reshape(1, 1)

    norm_src = jnp.sqrt(jnp.sum(z * z, axis=1, keepdims=True))
    norm_tgt = jnp.sqrt(jnp.sum(zq * zq, axis=1, keepdims=True))
    u = z / jnp.maximum(norm_src, _EPS)
    q = zq / jnp.maximum(norm_tgt, _EPS)
    w_ = u + q
    wn = jnp.sqrt(jnp.sum(w_ * w_, axis=1, keepdims=True))
    w_ = w_ / jnp.maximum(wn, _EPS)
    rotated = (z
               - 2.0 * jnp.sum(z * w_, axis=1, keepdims=True) * w_
               + 2.0 * jnp.sum(z * u, axis=1, keepdims=True) * q)
    scale = norm_tgt / jnp.maximum(norm_src, _EPS)
    out_ref[...] = (rotated * scale).reshape(_CB, _T, _D)


def _argmin_call(zc, v, g2, b2, fc, with_cbp):
    in_specs = [
        pl.BlockSpec((1, _M, _D), lambda i: (i // _TPB, i % _TPB, 0)),
        pl.BlockSpec((_D, _CD), lambda i: (0, 0)),
        pl.BlockSpec((1, _D), lambda i: (0, 0)),
        pl.BlockSpec((1, _D), lambda i: (0, 0)),
        pl.BlockSpec((_K, _CD), lambda i: (0, 0)),
    ]
    scratch = [
        pltpu.VMEM((_K, _D), jnp.float32),
        pltpu.VMEM((_M, _K), jnp.float32),
    ]
    params = pltpu.CompilerParams(dimension_semantics=("arbitrary",))
    if with_cbp:
        return pl.pallas_call(
            _argmin_step_cbp,
            grid=(_NBLKC,),
            in_specs=in_specs,
            out_specs=[
                pl.BlockSpec((1, 1, _M), lambda i: (i, 0, 0)),
                pl.BlockSpec((_K, 128), lambda i: (0, 0)),
            ],
            out_shape=[
                jax.ShapeDtypeStruct((_NBLKC, 1, _M), jnp.int32),
                jax.ShapeDtypeStruct((_K, 128), jnp.float32),
            ],
            scratch_shapes=scratch,
            compiler_params=params,
        )(zc, v, g2, b2, fc)
    idx3 = pl.pallas_call(
        _argmin_step,
        grid=(_NBLKC,),
        in_specs=in_specs,
        out_specs=pl.BlockSpec((1, 1, _M), lambda i: (i, 0, 0)),
        out_shape=jax.ShapeDtypeStruct((_NBLKC, 1, _M), jnp.int32),
        scratch_shapes=scratch,
        compiler_params=params,
    )(zc, v, g2, b2, fc)
    return idx3, None


_gather_call = functools.partial(
    pl.kernel,
    mesh=plsc.VectorSubcoreMesh(core_axis_name="c", subcore_axis_name="s"),
    out_type=jax.ShapeDtypeStruct((_CT, 128), jnp.float32),
    scratch_types=[
        pltpu.VMEM((_BPW,), jnp.int32),
        pltpu.VMEM((_BPW, 128), jnp.float32),
        pltpu.SemaphoreType.DMA,
    ],
)(_gather_sc)


def _rotate_call(zc, zq_raw):
    return pl.pallas_call(
        _rotate_step,
        in_specs=[
            pl.BlockSpec((_CB, _T, _D), lambda: (0, 0, 0)),
            pl.BlockSpec((_CT, 128), lambda: (0, 0)),
        ],
        out_specs=[
            pl.BlockSpec((_CB, _T, _D), lambda: (0, 0, 0)),
            pl.BlockSpec((1, 1), lambda: (0, 0)),
        ],
        out_shape=[
            jax.ShapeDtypeStruct((_CB, _T, _D), jnp.float32),
            jax.ShapeDtypeStruct((1, 1), jnp.float32),
        ],
    )(zc, zq_raw)


def kernel(z, v, g, b, frozen_codebook):
    g2 = g.reshape(1, _D)
    b2 = b.reshape(1, _D)

    idx_a, cbp = _argmin_call(z[:_CB], v, g2, b2, frozen_codebook, True)
    idx_b, _ = _argmin_call(z[_CB:], v, g2, b2, frozen_codebook, False)

    zq_a = _gather_call(cbp, idx_a.reshape(_CT))
    zq_b = _gather_call(cbp, idx_b.reshape(_CT))

    rot_a, loss_a = _rotate_call(z[:_CB], zq_a)
    rot_b, loss_b = _rotate_call(z[_CB:], zq_b)

    z_q = jnp.concatenate([rot_a, rot_b], axis=0)
    indices = jnp.concatenate(
        [idx_a.reshape(_CB, _T), idx_b.reshape(_CB, _T)], axis=0)
    commit_loss = (loss_a[0, 0] + loss_b[0, 0]) * (1.25 / (_N * _D))
    return (z_q, indices, commit_loss)


# speedup vs baseline: 1.0551x; 1.0551x over previous
import functools

import jax
import jax.numpy as jnp
from jax import lax
from jax.experimental import pallas as pl
from jax.experimental.pallas import tpu as pltpu
from jax.experimental.pallas import tpu_sc as plsc

_B, _T, _D = 16, 1024, 32
_K, _CD = 8192, 32
_N = _B * _T
_M = 512
_EPS = 1e-12

_CB = 16
_CT = _CB * _T
_NBLKC = _CT // _M
_TPB = _T // _M

_NC, _NS = 2, 16
_NW = _NC * _NS
_BPW = _CT // _NW


def _argmin_body(z_ref, v_ref, g_ref, b_ref, fc_ref,
                 idx_ref, cbp_ref, cb_ref, c2_ref):
    i = pl.program_id(0)

    @pl.when(i == 0)
    def _init():
        v = v_ref[...]
        vn = jnp.sqrt(jnp.sum(v * v, axis=1, keepdims=True))
        w = g_ref[...].reshape(_D, 1) * v / vn
        cb = jnp.dot(fc_ref[...], w.T,
                     preferred_element_type=jnp.float32) + b_ref[...]
        cb_ref[...] = cb
        c2 = jnp.sum(cb * cb, axis=1).reshape(1, _K)
        c2_ref[...] = jnp.broadcast_to(c2, (_M, _K))
        if cbp_ref is not None:
            cbp_ref[...] = jnp.pad(cb, ((0, 0), (0, 128 - _D)))

    cb = cb_ref[...]
    z = z_ref[...].reshape(_M, _D)
    z2 = jnp.sum(z * z, axis=1, keepdims=True)
    ncross = jnp.dot(-2.0 * z, cb.T,
                     preferred_element_type=jnp.float32)
    d2 = z2 + ncross + c2_ref[...]
    idx = jnp.argmin(d2, axis=1).astype(jnp.int32)
    idx_ref[...] = idx.reshape(1, 1, _M)


def _argmin_step_cbp(z_ref, v_ref, g_ref, b_ref, fc_ref,
                     idx_ref, cbp_ref, cb_ref, c2_ref):
    _argmin_body(z_ref, v_ref, g_ref, b_ref, fc_ref,
                 idx_ref, cbp_ref, cb_ref, c2_ref)


def _argmin_step(z_ref, v_ref, g_ref, b_ref, fc_ref,
                 idx_ref, cb_ref, c2_ref):
    _argmin_body(z_ref, v_ref, g_ref, b_ref, fc_ref,
                 idx_ref, None, cb_ref, c2_ref)


def _gather_sc(cbp_hbm, idx_hbm, out_hbm, idx_v, rows_v, sem):
    wid = lax.axis_index("s") * _NC + lax.axis_index("c")
    base = wid * _BPW
    pltpu.sync_copy(idx_hbm.at[pl.ds(base, _BPW)], idx_v)
    pltpu.async_copy(cbp_hbm.at[idx_v], rows_v, sem).wait()
    pltpu.sync_copy(rows_v, out_hbm.at[pl.ds(base, _BPW)])


_RCB = 8
_RCT = _RCB * _T
_NRBLK = _B // _RCB


def _rotate_step(z_ref, zq_ref, out_ref, loss_ref):
    z = z_ref[...].reshape(_RCT, _D)
    zq = zq_ref[:, :_D]
    diff = z - zq
    loss_ref[...] = jnp.sum(diff * diff).reshape(1, 1, 1)

    norm_src = jnp.sqrt(jnp.sum(z * z, axis=1, keepdims=True))
    norm_tgt = jnp.sqrt(jnp.sum(zq * zq, axis=1, keepdims=True))
    u = z / jnp.maximum(norm_src, _EPS)
    q = zq / jnp.maximum(norm_tgt, _EPS)
    w_ = u + q
    wn = jnp.sqrt(jnp.sum(w_ * w_, axis=1, keepdims=True))
    w_ = w_ / jnp.maximum(wn, _EPS)
    rotated = (z
               - 2.0 * jnp.sum(z * w_, axis=1, keepdims=True) * w_
               + 2.0 * jnp.sum(z * u, axis=1, keepdims=True) * q)
    scale = norm_tgt / jnp.maximum(norm_src, _EPS)
    out_ref[...] = (rotated * scale).reshape(_RCB, _T, _D)


def _argmin_call(zc, v, g2, b2, fc, with_cbp):
    in_specs = [
        pl.BlockSpec((1, _M, _D), lambda i: (i // _TPB, i % _TPB, 0)),
        pl.BlockSpec((_D, _CD), lambda i: (0, 0)),
        pl.BlockSpec((1, _D), lambda i: (0, 0)),
        pl.BlockSpec((1, _D), lambda i: (0, 0)),
        pl.BlockSpec((_K, _CD), lambda i: (0, 0)),
    ]
    scratch = [
        pltpu.VMEM((_K, _D), jnp.float32),
        pltpu.VMEM((_M, _K), jnp.float32),
    ]
    params = pltpu.CompilerParams(dimension_semantics=("arbitrary",))
    if with_cbp:
        return pl.pallas_call(
            _argmin_step_cbp,
            grid=(_NBLKC,),
            in_specs=in_specs,
            out_specs=[
                pl.BlockSpec((1, 1, _M), lambda i: (i, 0, 0)),
                pl.BlockSpec((_K, 128), lambda i: (0, 0)),
            ],
            out_shape=[
                jax.ShapeDtypeStruct((_NBLKC, 1, _M), jnp.int32),
                jax.ShapeDtypeStruct((_K, 128), jnp.float32),
            ],
            scratch_shapes=scratch,
            compiler_params=params,
        )(zc, v, g2, b2, fc)
    idx3 = pl.pallas_call(
        _argmin_step,
        grid=(_NBLKC,),
        in_specs=in_specs,
        out_specs=pl.BlockSpec((1, 1, _M), lambda i: (i, 0, 0)),
        out_shape=jax.ShapeDtypeStruct((_NBLKC, 1, _M), jnp.int32),
        scratch_shapes=scratch,
        compiler_params=params,
    )(zc, v, g2, b2, fc)
    return idx3, None


_gather_call = functools.partial(
    pl.kernel,
    mesh=plsc.VectorSubcoreMesh(core_axis_name="c", subcore_axis_name="s"),
    out_type=jax.ShapeDtypeStruct((_CT, 128), jnp.float32),
    scratch_types=[
        pltpu.VMEM((_BPW,), jnp.int32),
        pltpu.VMEM((_BPW, 128), jnp.float32),
        pltpu.SemaphoreType.DMA,
    ],
)(_gather_sc)


def _rotate_call(zc, zq_raw):
    return pl.pallas_call(
        _rotate_step,
        grid=(_NRBLK,),
        in_specs=[
            pl.BlockSpec((_RCB, _T, _D), lambda i: (i, 0, 0)),
            pl.BlockSpec((_RCT, 128), lambda i: (i, 0)),
        ],
        out_specs=[
            pl.BlockSpec((_RCB, _T, _D), lambda i: (i, 0, 0)),
            pl.BlockSpec((1, 1, 1), lambda i: (i, 0, 0)),
        ],
        out_shape=[
            jax.ShapeDtypeStruct((_B, _T, _D), jnp.float32),
            jax.ShapeDtypeStruct((_NRBLK, 1, 1), jnp.float32),
        ],
        compiler_params=pltpu.CompilerParams(
            dimension_semantics=("arbitrary",),
        ),
    )(zc, zq_raw)


def kernel(z, v, g, b, frozen_codebook):
    g2 = g.reshape(1, _D)
    b2 = b.reshape(1, _D)

    idx_a, cbp = _argmin_call(z, v, g2, b2, frozen_codebook, True)
    zq_a = _gather_call(cbp, idx_a.reshape(_CT))
    z_q, lparts = _rotate_call(z, zq_a)

    indices = idx_a.reshape(_B, _T)
    commit_loss = jnp.sum(lparts) * (1.25 / (_N * _D))
    return (z_q, indices, commit_loss)


# speedup vs baseline: 1.0564x; 1.0012x over previous
import functools

import jax
import jax.numpy as jnp
from jax import lax
from jax.experimental import pallas as pl
from jax.experimental.pallas import tpu as pltpu
from jax.experimental.pallas import tpu_sc as plsc

_B, _T, _D = 16, 1024, 32
_K, _CD = 8192, 32
_N = _B * _T
_M = 512
_EPS = 1e-12

_CB = 16
_CT = _CB * _T
_NBLKC = _CT // _M
_TPB = _T // _M

_NC, _NS = 2, 16
_NW = _NC * _NS
_BPW = _CT // _NW


def _argmin_body(z_ref, v_ref, g_ref, b_ref, fc_ref,
                 idx_ref, cbp_ref, cb_ref, c2_ref):
    i = pl.program_id(0)

    @pl.when(i == 0)
    def _init():
        v = v_ref[...]
        vn = jnp.sqrt(jnp.sum(v * v, axis=1, keepdims=True))
        w = g_ref[...].reshape(_D, 1) * v / vn
        cb = jnp.dot(fc_ref[...], w.T,
                     preferred_element_type=jnp.float32) + b_ref[...]
        cb_ref[...] = cb
        c2 = jnp.sum(cb * cb, axis=1).reshape(1, _K)
        c2_ref[...] = jnp.broadcast_to(c2, (_M, _K))
        if cbp_ref is not None:
            cbp_ref[...] = jnp.pad(cb, ((0, 0), (0, 128 - _D)))

    cb = cb_ref[...]
    z = z_ref[...].reshape(_M, _D)
    z2 = jnp.sum(z * z, axis=1, keepdims=True)
    ncross = jnp.dot(-2.0 * z, cb.T,
                     preferred_element_type=jnp.float32)
    d2 = z2 + ncross + c2_ref[...]
    idx = jnp.argmin(d2, axis=1).astype(jnp.int32)
    idx_ref[...] = idx


def _argmin_step_cbp(z_ref, v_ref, g_ref, b_ref, fc_ref,
                     idx_ref, cbp_ref, cb_ref, c2_ref):
    _argmin_body(z_ref, v_ref, g_ref, b_ref, fc_ref,
                 idx_ref, cbp_ref, cb_ref, c2_ref)


def _argmin_step(z_ref, v_ref, g_ref, b_ref, fc_ref,
                 idx_ref, cb_ref, c2_ref):
    _argmin_body(z_ref, v_ref, g_ref, b_ref, fc_ref,
                 idx_ref, None, cb_ref, c2_ref)


def _gather_sc(cbp_hbm, idx_hbm, out_hbm, idx_v, rows_v, sem):
    wid = lax.axis_index("s") * _NC + lax.axis_index("c")
    base = wid * _BPW
    pltpu.sync_copy(idx_hbm.at[pl.ds(base, _BPW)], idx_v)
    pltpu.async_copy(cbp_hbm.at[idx_v], rows_v, sem).wait()
    pltpu.sync_copy(rows_v, out_hbm.at[pl.ds(base, _BPW)])


_RCB = 8
_RCT = _RCB * _T
_NRBLK = _B // _RCB


def _rotate_step(z_ref, zq_ref, out_ref, loss_ref):
    z = z_ref[...].reshape(_RCT, _D)
    zq = zq_ref[:, :_D]
    diff = z - zq
    loss_ref[...] = jnp.sum(diff * diff).reshape(1, 1, 1)

    norm_src = jnp.sqrt(jnp.sum(z * z, axis=1, keepdims=True))
    norm_tgt = jnp.sqrt(jnp.sum(zq * zq, axis=1, keepdims=True))
    u = z / jnp.maximum(norm_src, _EPS)
    q = zq / jnp.maximum(norm_tgt, _EPS)
    w_ = u + q
    wn = jnp.sqrt(jnp.sum(w_ * w_, axis=1, keepdims=True))
    w_ = w_ / jnp.maximum(wn, _EPS)
    rotated = (z
               - 2.0 * jnp.sum(z * w_, axis=1, keepdims=True) * w_
               + 2.0 * jnp.sum(z * u, axis=1, keepdims=True) * q)
    scale = norm_tgt / jnp.maximum(norm_src, _EPS)
    out_ref[...] = (rotated * scale).reshape(_RCB, _T, _D)


def _argmin_call(zc, v, g2, b2, fc, with_cbp):
    in_specs = [
        pl.BlockSpec((1, _M, _D), lambda i: (i // _TPB, i % _TPB, 0)),
        pl.BlockSpec((_D, _CD), lambda i: (0, 0)),
        pl.BlockSpec((1, _D), lambda i: (0, 0)),
        pl.BlockSpec((1, _D), lambda i: (0, 0)),
        pl.BlockSpec((_K, _CD), lambda i: (0, 0)),
    ]
    scratch = [
        pltpu.VMEM((_K, _D), jnp.float32),
        pltpu.VMEM((_M, _K), jnp.float32),
    ]
    params = pltpu.CompilerParams(dimension_semantics=("arbitrary",))
    if with_cbp:
        return pl.pallas_call(
            _argmin_step_cbp,
            grid=(_NBLKC,),
            in_specs=in_specs,
            out_specs=[
                pl.BlockSpec((_M,), lambda i: (i,)),
                pl.BlockSpec((_K, 128), lambda i: (0, 0)),
            ],
            out_shape=[
                jax.ShapeDtypeStruct((_CT, ), jnp.int32),
                jax.ShapeDtypeStruct((_K, 128), jnp.float32),
            ],
            scratch_shapes=scratch,
            compiler_params=params,
        )(zc, v, g2, b2, fc)
    idx1 = pl.pallas_call(
        _argmin_step,
        grid=(_NBLKC,),
        in_specs=in_specs,
        out_specs=pl.BlockSpec((_M,), lambda i: (i,)),
        out_shape=jax.ShapeDtypeStruct((_CT,), jnp.int32),
        scratch_shapes=scratch,
        compiler_params=params,
    )(zc, v, g2, b2, fc)
    return idx1, None


_gather_call = functools.partial(
    pl.kernel,
    mesh=plsc.VectorSubcoreMesh(core_axis_name="c", subcore_axis_name="s"),
    out_type=jax.ShapeDtypeStruct((_CT, 128), jnp.float32),
    scratch_types=[
        pltpu.VMEM((_BPW,), jnp.int32),
        pltpu.VMEM((_BPW, 128), jnp.float32),
        pltpu.SemaphoreType.DMA,
    ],
)(_gather_sc)


def _rotate_call(zc, zq_raw):
    return pl.pallas_call(
        _rotate_step,
        grid=(_NRBLK,),
        in_specs=[
            pl.BlockSpec((_RCB, _T, _D), lambda i: (i, 0, 0)),
            pl.BlockSpec((_RCT, 128), lambda i: (i, 0)),
        ],
        out_specs=[
            pl.BlockSpec((_RCB, _T, _D), lambda i: (i, 0, 0)),
            pl.BlockSpec((1, 1, 1), lambda i: (i, 0, 0)),
        ],
        out_shape=[
            jax.ShapeDtypeStruct((_B, _T, _D), jnp.float32),
            jax.ShapeDtypeStruct((_NRBLK, 1, 1), jnp.float32),
        ],
        compiler_params=pltpu.CompilerParams(
            dimension_semantics=("arbitrary",),
        ),
    )(zc, zq_raw)


def kernel(z, v, g, b, frozen_codebook):
    g2 = g.reshape(1, _D)
    b2 = b.reshape(1, _D)

    idx_a, cbp = _argmin_call(z, v, g2, b2, frozen_codebook, True)
    zq_a = _gather_call(cbp, idx_a)
    z_q, lparts = _rotate_call(z, zq_a)

    indices = idx_a.reshape(_B, _T)
    commit_loss = jnp.sum(lparts) * (1.25 / (_N * _D))
    return (z_q, indices, commit_loss)
